# baseline (speedup 1.0000x reference)
import functools

import jax
import jax.numpy as jnp
from jax import lax
from jax.experimental import pallas as pl
from jax.experimental.pallas import tpu as pltpu
from jax.experimental.pallas import tpu_sc as plsc

NC = 2
NS = 16
NW = NC * NS
LANES = 16
NBUF = 4


def _make_sc_kernel(B, D, b_per_w):
    n_chunks = b_per_w // LANES
    mesh = plsc.VectorSubcoreMesh(core_axis_name="c", subcore_axis_name="s")

    @functools.partial(
        pl.kernel,
        mesh=mesh,
        out_type=jax.ShapeDtypeStruct((NW, LANES), jnp.float32),
        compiler_params=pltpu.CompilerParams(use_tc_tiling_on_sc=True,
                                             needs_layout_passes=False),
        scratch_types=[
            pltpu.VMEM((b_per_w,), jnp.int32),
            pltpu.VMEM((b_per_w, D), jnp.float32),
            pltpu.VMEM((NBUF, D, 128), jnp.float32),
            pltpu.VMEM((LANES,), jnp.float32),
            pltpu.SemaphoreType.DMA,
            pltpu.SemaphoreType.DMA((NBUF,)),
        ],
    )
    def sc_kernel(emb_hbm, idx_hbm, tblT_hbm, out_hbm,
                  idx_v, emb_v, blk_v, res_v, sem_e, sem_g):
        wid = lax.axis_index("s") * NC + lax.axis_index("c")
        base = wid * b_per_w

        emb_cp = pltpu.async_copy(emb_hbm.at[pl.ds(base, b_per_w)], emb_v, sem_e)
        pltpu.sync_copy(idx_hbm.at[pl.ds(base, b_per_w)], idx_v)

        def fire(l, slot):
            col = pl.multiple_of((l >> 7) << 7, 128)
            pltpu.async_copy(tblT_hbm.at[:, pl.ds(col, 128)], blk_v.at[slot],
                             sem_g.at[slot])

        v0 = idx_v[pl.ds(0, LANES)]
        for j in range(NBUF):
            fire(v0[j], j)
        emb_cp.wait()

        lane = lax.iota(jnp.int32, LANES)

        def body(g, accs):
            out = list(accs)
            vc = idx_v[pl.ds(g * LANES, LANES)]
            gn = jnp.minimum(g + 1, n_chunks - 1)
            vn = idx_v[pl.ds(gn * LANES, LANES)]
            not_last = g + 1 < n_chunks
            for h in range(LANES // NBUF):
                for j in range(NBUF):
                    l = vc[NBUF * h + j]
                    sub = jnp.full((LANES,), l & 127, jnp.int32)
                    r = g * LANES + NBUF * h + j
                    pltpu.make_async_copy(
                        tblT_hbm.at[:, pl.ds(0, 128)], blk_v.at[j],
                        sem_g.at[j]).wait()
                    for f in range(D // LANES):
                        sl = pl.ds(f * LANES, LANES)
                        c = plsc.load_gather(
                            blk_v.at[j], [lane + f * LANES, sub])
                        d = emb_v[r, sl] - c
                        out[f] = out[f] + d * d
                    nxt = NBUF * (h + 1) + j
                    if nxt < LANES:
                        fire(vc[nxt], j)
                    else:
                        ln = vn[j]

                        @pl.when(not_last)
                        def _():
                            fire(ln, j)
            return tuple(out)

        zero = jnp.zeros((LANES,), jnp.float32)
        accs = lax.fori_loop(0, n_chunks, body, (zero,) * (D // LANES))
        total = accs[0]
        for a in accs[1:]:
            total = total + a
        res_v[...] = total
        pltpu.sync_copy(res_v, out_hbm.at[wid])

    return sc_kernel


def kernel(embedding_batch, label_batch, class_centers):
    B, D = embedding_batch.shape
    order = jnp.argsort(label_batch)
    slab = label_batch[order].astype(jnp.int32)
    emb_s = embedding_batch[order]
    sc_kernel = _make_sc_kernel(B, D, B // NW)
    partials = sc_kernel(emb_s, slab, class_centers.T)
    return jnp.sum(partials) / B


# speedup vs baseline: 1.1634x; 1.1634x over previous
import functools

import jax
import jax.numpy as jnp
from jax import lax
from jax.experimental import pallas as pl
from jax.experimental.pallas import tpu as pltpu
from jax.experimental.pallas import tpu_sc as plsc

NC = 2
NS = 16
NW = NC * NS
LANES = 16
NBUF = 4
UBLK_LEN = 536


def _make_sc_kernel(B, D, b_per_w):
    n_chunks = b_per_w // LANES
    mesh = plsc.VectorSubcoreMesh(core_axis_name="c", subcore_axis_name="s")

    @functools.partial(
        pl.kernel,
        mesh=mesh,
        out_type=jax.ShapeDtypeStruct((NW, LANES), jnp.float32),
        compiler_params=pltpu.CompilerParams(use_tc_tiling_on_sc=True,
                                             needs_layout_passes=False),
        scratch_types=[
            pltpu.VMEM((NW,), jnp.int32),
            pltpu.VMEM((b_per_w,), jnp.int32),
            pltpu.VMEM((UBLK_LEN,), jnp.int32),
            pltpu.VMEM((b_per_w, D), jnp.float32),
            pltpu.VMEM((NBUF * D, 128), jnp.float32),
            pltpu.VMEM((LANES,), jnp.float32),
            pltpu.SemaphoreType.DMA,
            pltpu.SemaphoreType.DMA((NBUF,)),
        ],
    )
    def sc_kernel(emb_hbm, slab_hbm, ublk_hbm, wkb_hbm, tblT_hbm,
                  out_hbm, wkb_v, idx_v, ublk_v, emb_v, blk_v, res_v,
                  sem_e, sem_g):
        wid = lax.axis_index("s") * NC + lax.axis_index("c")
        base = wid * b_per_w

        pltpu.sync_copy(wkb_hbm, wkb_v)
        k0 = plsc.load_gather(wkb_v, [jnp.full((LANES,), wid, jnp.int32)])[0]
        k0_al = pl.multiple_of((k0 >> 3) << 3, 8)
        koff = k0 - k0_al
        pltpu.sync_copy(slab_hbm.at[pl.ds(base, b_per_w)], idx_v)
        pltpu.sync_copy(ublk_hbm.at[pl.ds(k0_al, UBLK_LEN)], ublk_v)

        def ublk_at(i):
            return plsc.load_gather(
                ublk_v, [jnp.full((LANES,), koff + i, jnp.int32)])[0]

        emb_cp = pltpu.async_copy(emb_hbm.at[pl.ds(base, b_per_w)], emb_v,
                                  sem_e)

        def fire(blockid, slot):
            slot_m = slot % NBUF
            col = pl.multiple_of(blockid * 128, 128)
            row = pl.multiple_of(slot_m * D, D)
            pltpu.async_copy(tblT_hbm.at[:, pl.ds(col, 128)],
                             blk_v.at[pl.ds(row, D)], sem_g.at[slot_m])

        for t in range(NBUF - 1):
            fire(ublk_at(t), t)
        emb_cp.wait()

        lane = lax.iota(jnp.int32, LANES)

        def body(g, carry):
            carry_b, lk, a0, a1, a2, a3 = carry
            out = [a0, a1, a2, a3]
            vc = idx_v[pl.ds(g * LANES, LANES)]
            for j in range(LANES):
                l = vc[j]
                b = l >> 7
                is_new = b != carry_b
                carry_b = b
                lk = lk + is_new.astype(jnp.int32)

                for s in range(NBUF):
                    @pl.when(is_new & (lk % NBUF == s))
                    def _(s=s):
                        pltpu.make_async_copy(
                            tblT_hbm.at[:, pl.ds(0, 128)],
                            blk_v.at[pl.ds(s * D, D)],
                            sem_g.at[s]).wait()
                        fire(ublk_at(lk + NBUF - 1), (s - 1) % NBUF)

                slot = lk % NBUF
                sub = jnp.full((LANES,), l & 127, jnp.int32)
                r = g * LANES + j
                for f in range(D // LANES):
                    sl = pl.ds(f * LANES, LANES)
                    cvec = plsc.load_gather(
                        blk_v, [slot * D + f * LANES + lane, sub])
                    d = emb_v[r, sl] - cvec
                    out[f] = out[f] + d * d
            return (carry_b, lk, out[0], out[1], out[2], out[3])

        zero = jnp.zeros((LANES,), jnp.float32)
        init = (jnp.int32(-1), jnp.int32(-1), zero, zero, zero, zero)
        _, lk_f, a0, a1, a2, a3 = lax.fori_loop(0, n_chunks, body, init)
        for s in range(NBUF):
            @pl.when(lk_f % NBUF != s)
            def _(s=s):
                pltpu.make_async_copy(
                    tblT_hbm.at[:, pl.ds(0, 128)],
                    blk_v.at[pl.ds(s * D, D)],
                    sem_g.at[s]).wait()
        res_v[...] = a0 + a1 + a2 + a3
        pltpu.sync_copy(res_v, out_hbm.at[wid])

    return sc_kernel


def kernel(embedding_batch, label_batch, class_centers):
    B, D = embedding_batch.shape
    b_per_w = B // NW

    labels = label_batch.astype(jnp.int32)
    order = jnp.argsort(labels).astype(jnp.int32)
    slab = jnp.take(labels, order)
    blocks = slab >> 7
    inew = jnp.concatenate(
        [jnp.ones((1,), jnp.int32),
         (blocks[1:] != blocks[:-1]).astype(jnp.int32)])
    snum = jnp.cumsum(inew) - 1
    ustart = jnp.nonzero(inew, size=B, fill_value=B - 1)[0].astype(jnp.int32)
    ublocks = jnp.take(blocks, ustart)
    ublocks = jnp.concatenate(
        [ublocks, jnp.full((UBLK_LEN + 8, ), 0, jnp.int32)])
    wkb = jnp.take(snum, jnp.arange(NW, dtype=jnp.int32) * b_per_w
                   ).astype(jnp.int32)

    emb_s = jnp.take(embedding_batch, order, axis=0)
    sc_kernel = _make_sc_kernel(B, D, b_per_w)
    partials = sc_kernel(emb_s, slab, ublocks, wkb, class_centers.T)
    return jnp.sum(partials) / B
